# Initial kernel scaffold; baseline (speedup 1.0000x reference)
import jax
import jax.numpy as jnp
from jax.experimental import pallas as pl


def kernel(x, y, pe_h, pe_w):
    raise NotImplementedError("write your pallas kernel here")


# speedup vs baseline: 3.7155x; 3.7155x over previous
import jax
import jax.numpy as jnp
from jax import lax
from jax.experimental import pallas as pl
from jax.experimental.pallas import tpu as pltpu
from jax.experimental.pallas import tpu_sc as plsc

D_HALF = 64
D = 2 * D_HALF
BATCH = 16384

_info = plsc.get_sparse_core_info()
_NC, _NS = _info.num_cores, _info.num_subcores
_NW = _NC * _NS
_B_PER_W = BATCH // _NW
_CHUNK = 128
_N_CHUNKS = _B_PER_W // _CHUNK


def _pe_body(x_hbm, y_hbm, peh_hbm, pew_hbm, out_hbm, idx_x, idx_y, rows, sem):
    wid = lax.axis_index("s") * _NC + lax.axis_index("c")
    base = wid * _B_PER_W
    pltpu.sync_copy(x_hbm.at[pl.ds(base, _B_PER_W)], idx_x)
    pltpu.sync_copy(y_hbm.at[pl.ds(base, _B_PER_W)], idx_y)
    waves = []
    for c in range(_N_CHUNKS):
        off = c * _CHUNK
        waves.append(pltpu.async_copy(
            pew_hbm.at[idx_x.at[pl.ds(off, _CHUNK)]],
            rows.at[pl.ds(off, _CHUNK)], sem))
    for w in waves:
        w.wait()
    waves = []
    for c in range(_N_CHUNKS):
        off = c * _CHUNK
        waves.append(pltpu.async_copy(
            peh_hbm.at[idx_y.at[pl.ds(off, _CHUNK)]],
            rows.at[pl.ds(off, _CHUNK)], sem, add=True))
    for w in waves:
        w.wait()
    pltpu.sync_copy(rows, out_hbm.at[pl.ds(base, _B_PER_W)])


@jax.jit
def _pe_kernel(x, y, peh_wide, pew_wide):
    mesh = plsc.VectorSubcoreMesh(core_axis_name="c", subcore_axis_name="s")
    return pl.kernel(
        _pe_body,
        out_type=jax.ShapeDtypeStruct((BATCH, D), jnp.float32),
        mesh=mesh,
        scratch_types=[
            pltpu.VMEM((_B_PER_W,), jnp.int32),
            pltpu.VMEM((_B_PER_W,), jnp.int32),
            pltpu.VMEM((_B_PER_W, D), jnp.float32),
            pltpu.SemaphoreType.DMA,
        ],
    )(x, y, peh_wide, pew_wide)


def kernel(x, y, pe_h, pe_w):
    x = x.astype(jnp.int32)
    y = y.astype(jnp.int32)
    zeros = jnp.zeros_like(pe_w)
    pew_wide = jnp.concatenate([pe_w, zeros], axis=1)
    peh_wide = jnp.concatenate([zeros, pe_h], axis=1)
    return _pe_kernel(x, y, peh_wide, pew_wide)
